# Initial kernel scaffold; baseline (speedup 1.0000x reference)
import jax
import jax.numpy as jnp
from jax.experimental import pallas as pl


def kernel(depth_mask_3C):
    raise NotImplementedError("write your pallas kernel here")


# speedup vs baseline: 43.7928x; 43.7928x over previous
import functools

import numpy as np
import jax
import jax.numpy as jnp
from jax import lax
from jax.experimental import pallas as pl
from jax.experimental.pallas import tpu as pltpu
from jax.experimental.pallas import tpu_sc as plsc

B, H, W = 64, 150, 200
P, K = 8, 512
N = H * W
NV = N // 16
OUTCOLS = P * (K + 1)
OUT_PAD = 12320
C0 = int(np.float32(3.0).view(np.int32))
NB1 = 896
NB2 = 128
NB3 = 128

_hfov = float(np.deg2rad(81.0))
_vfov = float(np.deg2rad(59.0))
_fx = W / (2.0 * np.tan(_hfov / 2.0))
_fy = H / (2.0 * np.tan(_vfov / 2.0))
INV_FX = float(1.0 / _fx)
INV_FY = float(1.0 / _fy)

_i32 = jnp.int32
_f32 = jnp.float32


def _io():
    return lax.iota(_i32, 16)


def _vgather(x, idx):
    return lax.gather(
        x, idx[:, None],
        dimension_numbers=lax.GatherDimensionNumbers(
            offset_dims=(), collapsed_slice_dims=(0,), start_index_map=(0,)),
        slice_sizes=(1,),
        mode=lax.GatherScatterMode.PROMISE_IN_BOUNDS)


def _last_lane(x):
    return _vgather(x, jnp.full((16,), 15, _i32))


def _bsearch(ref, base, r, iters, nbins):
    lo = jnp.zeros((16,), _i32)
    hi = jnp.full((16,), nbins, _i32)
    for _ in range(iters):
        mid = (lo + hi) >> 1
        g = plsc.load_gather(ref, [base + jnp.minimum(mid, nbins - 1)])
        pred = g > r
        hi = jnp.where(pred, mid, hi)
        lo = jnp.where(pred, lo, mid + 1)
    return jnp.minimum(lo, nbins - 1)


def _make_kernel(interpret=False):
    mesh = plsc.VectorSubcoreMesh(core_axis_name="c", subcore_axis_name="s",
                                  num_cores=2, num_subcores=16)

    @functools.partial(
        pl.kernel,
        mesh=mesh,
        out_type=jax.ShapeDtypeStruct((B, OUT_PAD), _f32),
        scratch_types=[
            pltpu.VMEM((N,), _f32),
            pltpu.VMEM((N,), _f32),
            pltpu.VMEM((N,), _i32),
            pltpu.VMEM((P * NB1,), _i32),
            pltpu.VMEM((32 * NB2,), _i32),
            pltpu.VMEM((32 * NB3,), _i32),
            pltpu.VMEM((OUT_PAD,), _f32),
            pltpu.VMEM((32,), _i32),
            pltpu.VMEM((32,), _i32),
            pltpu.VMEM((16,), _f32),
            pltpu.VMEM((16,), _f32),
        ],
        compiler_params=pltpu.CompilerParams(needs_layout_passes=False),
        interpret=interpret,
    )
    def sc_kernel(in_hbm, out_hbm, dbuf, ibuf, key_v, h1, h2, h3, stage,
                  t1_v, t12_v, lb_v, ub_v):
        wid = lax.axis_index("s") * 2 + lax.axis_index("c")
        io = _io()
        zeros16 = jnp.zeros((16,), _i32)
        ones16 = jnp.ones((16,), _i32)

        for bi in range(2):
            b = wid * 2 + bi
            pltpu.sync_copy(in_hbm.at[b, 0], dbuf)
            pltpu.sync_copy(in_hbm.at[b, 1], ibuf)

            def pack_body(i, _):
                s = i * 16
                d = dbuf[pl.ds(s, 16)]
                nd = ibuf[pl.ds(s, 16)]
                pid = (nd + 0.5).astype(_i32)
                off = lax.bitcast_convert_type(d, _i32) - C0
                off = jnp.minimum(off, (1 << 24) - (1 << 14))
                valid = (pid >= 1) & (pid <= P) & (d > 3.0)
                key_v[pl.ds(s, 16)] = jnp.where(
                    valid, ((pid - 1) << 24) | off, -1)
                return 0

            lax.fori_loop(0, NV, pack_body, 0)

            def z1(i, _):
                h1[pl.ds(i * 16, 16)] = zeros16
                return 0

            def z2(i, _):
                h2[pl.ds(i * 16, 16)] = zeros16
                h3[pl.ds(i * 16, 16)] = zeros16
                return 0

            lax.fori_loop(0, P * NB1 // 16, z1, 0)
            lax.fori_loop(0, 32 * NB2 // 16, z2, 0)

            def scanA(i, _):
                k = key_v[pl.ds(i * 16, 16)]
                m = k >= 0
                pi = (k >> 24) & 7
                bin1 = jnp.minimum((k & 0xFFFFFF) >> 14, NB1 - 1)
                plsc.addupdate_scatter(h1, [pi * NB1 + bin1], ones16, mask=m)
                return 0

            lax.fori_loop(0, NV, scanA, 0)

            for p in range(P):
                def cs1(j, carry, p=p):
                    sl = pl.ds(p * NB1 + j * 16, 16)
                    s = plsc.cumsum(h1[sl]) + carry
                    h1[sl] = s
                    return _last_lane(s)

                lax.fori_loop(0, NB1 // 16, cs1, zeros16)

            pl8 = jnp.minimum(io, 7)
            n = plsc.load_gather(h1, [pl8 * NB1 + (NB1 - 1)])
            nn = jnp.maximum(n, 1)
            nm1f = (nn - 1).astype(_f32)
            pos25 = 0.25 * nm1f
            pos75 = 0.75 * nm1f
            i0_25 = pos25.astype(_i32)
            i0_75 = pos75.astype(_i32)
            frac25 = pos25 - i0_25.astype(_f32)
            frac75 = pos75 - i0_75.astype(_f32)
            i1_25 = jnp.minimum(i0_25 + 1, nn - 1)
            i1_75 = jnp.minimum(i0_75 + 1, nn - 1)
            lane_p = io & 7
            rA = jnp.where(io < 8, i0_25, _vgather(i1_25, lane_p))
            rB = jnp.where(io < 8, i0_75, _vgather(i1_75, lane_p))

            pbase = lane_p * NB1
            b1A = _bsearch(h1, pbase, rA, 10, NB1)
            b1B = _bsearch(h1, pbase, rB, 10, NB1)
            gpA = plsc.load_gather(h1, [pbase + jnp.maximum(b1A - 1, 0)])
            gpB = plsc.load_gather(h1, [pbase + jnp.maximum(b1B - 1, 0)])
            rA2 = rA - jnp.where(b1A > 0, gpA, 0)
            rB2 = rB - jnp.where(b1B > 0, gpB, 0)
            t1_v[pl.ds(0, 16)] = b1A
            t1_v[pl.ds(16, 16)] = b1B

            def scanB(i, _):
                k = key_v[pl.ds(i * 16, 16)]
                m = k >= 0
                pi = (k >> 24) & 7
                off = k & 0xFFFFFF
                b1 = off >> 14
                sub = (off >> 7) & (NB2 - 1)
                for c in range(4):
                    tb = plsc.load_gather(t1_v, [pi + 8 * c])
                    mc = m & (b1 == tb)
                    plsc.addupdate_scatter(
                        h2, [(pi + 8 * c) * NB2 + sub], ones16, mask=mc)
                return 0

            lax.fori_loop(0, NV, scanB, 0)

            def cs2(ch, _):
                base = ch * NB2
                carry = zeros16
                for kk in range(NB2 // 16):
                    sl = pl.ds(base + kk * 16, 16)
                    s = plsc.cumsum(h2[sl]) + carry
                    h2[sl] = s
                    carry = _last_lane(s)
                return 0

            lax.fori_loop(0, 32, cs2, 0)

            cbaseA = io * NB2
            cbaseB = (16 + io) * NB2
            b2A = _bsearch(h2, cbaseA, rA2, 7, NB2)
            b2B = _bsearch(h2, cbaseB, rB2, 7, NB2)
            gp2A = plsc.load_gather(h2, [cbaseA + jnp.maximum(b2A - 1, 0)])
            gp2B = plsc.load_gather(h2, [cbaseB + jnp.maximum(b2B - 1, 0)])
            rA3 = rA2 - jnp.where(b2A > 0, gp2A, 0)
            rB3 = rB2 - jnp.where(b2B > 0, gp2B, 0)
            t12A = (b1A << 7) | b2A
            t12B = (b1B << 7) | b2B
            t12_v[pl.ds(0, 16)] = t12A
            t12_v[pl.ds(16, 16)] = t12B

            def scanC(i, _):
                k = key_v[pl.ds(i * 16, 16)]
                m = k >= 0
                pi = (k >> 24) & 7
                off = k & 0xFFFFFF
                hi17 = off >> 7
                sub = off & (NB3 - 1)
                for c in range(4):
                    tb = plsc.load_gather(t12_v, [pi + 8 * c])
                    mc = m & (hi17 == tb)
                    plsc.addupdate_scatter(
                        h3, [(pi + 8 * c) * NB3 + sub], ones16, mask=mc)
                return 0

            lax.fori_loop(0, NV, scanC, 0)

            def cs3(ch, _):
                base = ch * NB3
                carry = zeros16
                for kk in range(NB3 // 16):
                    sl = pl.ds(base + kk * 16, 16)
                    s = plsc.cumsum(h3[sl]) + carry
                    h3[sl] = s
                    carry = _last_lane(s)
                return 0

            lax.fori_loop(0, 32, cs3, 0)

            b3A = _bsearch(h3, cbaseA, rA3, 7, NB3)
            b3B = _bsearch(h3, cbaseB, rB3, 7, NB3)
            offA = (t12A << 7) | b3A
            offB = (t12B << 7) | b3B
            valA = lax.bitcast_convert_type(offA + C0, _f32)
            valB = lax.bitcast_convert_type(offB + C0, _f32)

            hi_idx = jnp.minimum(io + 8, 15)
            v0q1 = valA
            v1q1 = _vgather(valA, hi_idx)
            v0q3 = valB
            v1q3 = _vgather(valB, hi_idx)
            q1 = v0q1 * (1.0 - frac25) + v1q1 * frac25
            q3 = v0q3 * (1.0 - frac75) + v1q3 * frac75
            iqr = q3 - q1
            lb = q1 - 1.5 * iqr
            ub = q3 + 1.5 * iqr
            npos = n > 0
            posinf = jnp.full((16,), jnp.inf, _f32)
            lb_v[...] = jnp.where(npos, lb, posinf)
            ub_v[...] = jnp.where(npos, ub, -posinf)

            zf16 = jnp.zeros((16,), _f32)

            def zs(i, _):
                stage[pl.ds(i * 16, 16)] = zf16
                return 0

            lax.fori_loop(0, OUT_PAD // 16, zs, 0)

            def emit(i, counters):
                s = i * 16
                k = key_v[pl.ds(s, 16)]
                m = k >= 0
                pi = (k >> 24) & 7
                z = lax.bitcast_convert_type((k & 0xFFFFFF) + C0, _f32)
                lbg = plsc.load_gather(lb_v, [pi])
                ubg = plsc.load_gather(ub_v, [pi])
                filt = m & (z >= lbg) & (z <= ubg)
                sh = (pi & 3) << 3
                onev = jnp.left_shift(ones16, sh)
                low = pi < 4
                enc1 = jnp.where(filt & low, onev, 0)
                enc2 = jnp.where(filt & (~low), onev, 0)
                cs1v = plsc.cumsum(enc1)
                cs2v = plsc.cumsum(enc2)
                cssel = jnp.where(low, cs1v, cs2v)
                within = (cssel >> sh) & 255
                basec = _vgather(counters, pi)
                rank = basec + within - 1
                emitm = filt & (rank < K)
                col = pi * (K + 1) + rank
                fpix = (s + io).astype(_f32)
                rowf = ((fpix + 0.5) * 0.005).astype(_i32).astype(_f32)
                colf = fpix - rowf * 200.0
                xch = (colf - 100.0) * INV_FX
                ych = (rowf - 75.0) * INV_FY
                plsc.store_scatter(stage, [col], xch * z, mask=emitm)
                plsc.store_scatter(stage, [col + OUTCOLS], ych * z, mask=emitm)
                plsc.store_scatter(stage, [col + 2 * OUTCOLS], z, mask=emitm)
                tot = jnp.where(io < 4, _last_lane(cs1v), _last_lane(cs2v))
                inc = (tot >> ((io & 3) << 3)) & 255
                return counters + jnp.where(io < 8, inc, 0)

            counters = lax.fori_loop(0, NV, emit, zeros16)

            flagv = jnp.where(counters > 0, 1.0, 0.0).astype(_f32)
            plsc.store_scatter(stage, [pl8 * (K + 1) + K], flagv, mask=io < 8)

            pltpu.sync_copy(stage, out_hbm.at[b])

    return sc_kernel


_kernel_fn = _make_kernel(interpret=False)


@jax.jit
def kernel(depth_mask_3C):
    x = depth_mask_3C.reshape(B, 3, N)
    outp = _kernel_fn(x)
    return outp[:, :3 * OUTCOLS].reshape(B, 3, OUTCOLS)


# speedup vs baseline: 111.8733x; 2.5546x over previous
import functools

import numpy as np
import jax
import jax.numpy as jnp
from jax import lax
from jax.experimental import pallas as pl
from jax.experimental.pallas import tpu as pltpu
from jax.experimental.pallas import tpu_sc as plsc

B, H, W = 64, 150, 200
P, K = 8, 512
N = H * W
NV = N // 16
OUTCOLS = P * (K + 1)
OUT_PAD = 12320
C0 = int(np.float32(3.0).view(np.int32))
NB1 = 896
NB2 = 128
NB3 = 128

_hfov = float(np.deg2rad(81.0))
_vfov = float(np.deg2rad(59.0))
_fx = W / (2.0 * np.tan(_hfov / 2.0))
_fy = H / (2.0 * np.tan(_vfov / 2.0))
INV_FX = float(1.0 / _fx)
INV_FY = float(1.0 / _fy)

_i32 = jnp.int32
_f32 = jnp.float32


def _io():
    return lax.iota(_i32, 16)


def _vgather(x, idx):
    return lax.gather(
        x, idx[:, None],
        dimension_numbers=lax.GatherDimensionNumbers(
            offset_dims=(), collapsed_slice_dims=(0,), start_index_map=(0,)),
        slice_sizes=(1,),
        mode=lax.GatherScatterMode.PROMISE_IN_BOUNDS)


def _last_lane(x):
    return _vgather(x, jnp.full((16,), 15, _i32))


def _bsearch(ref, base, r, iters, nbins):
    lo = jnp.zeros((16,), _i32)
    hi = jnp.full((16,), nbins, _i32)
    for _ in range(iters):
        mid = (lo + hi) >> 1
        g = plsc.load_gather(ref, [base + jnp.minimum(mid, nbins - 1)])
        pred = g > r
        hi = jnp.where(pred, mid, hi)
        lo = jnp.where(pred, lo, mid + 1)
    return jnp.minimum(lo, nbins - 1)


def _make_kernel(interpret=False):
    mesh = plsc.VectorSubcoreMesh(core_axis_name="c", subcore_axis_name="s",
                                  num_cores=2, num_subcores=16)

    @functools.partial(
        pl.kernel,
        mesh=mesh,
        out_type=jax.ShapeDtypeStruct((B, OUT_PAD), _f32),
        scratch_types=[
            pltpu.VMEM((N,), _f32),
            pltpu.VMEM((N,), _f32),
            pltpu.VMEM((N,), _i32),
            pltpu.VMEM((P * NB1,), _i32),
            pltpu.VMEM((32 * NB2,), _i32),
            pltpu.VMEM((32 * NB3,), _i32),
            pltpu.VMEM((OUT_PAD,), _f32),
            pltpu.VMEM((32,), _i32),
            pltpu.VMEM((32,), _i32),
            pltpu.VMEM((16,), _f32),
            pltpu.VMEM((16,), _f32),
            pltpu.SemaphoreType.DMA,
            pltpu.SemaphoreType.DMA,
        ],
        compiler_params=pltpu.CompilerParams(needs_layout_passes=False),
        interpret=interpret,
    )
    def sc_kernel(in_hbm, out_hbm, dbuf, ibuf, key_v, h1, h2, h3, stage,
                  t1_v, t12_v, lb_v, ub_v, insem, outsem):
        wid = lax.axis_index("s") * 2 + lax.axis_index("c")
        io = _io()
        zeros16 = jnp.zeros((16,), _i32)
        ones16 = jnp.ones((16,), _i32)
        zf16 = jnp.zeros((16,), _f32)

        pltpu.sync_copy(in_hbm.at[wid * 2, 0], dbuf)
        pltpu.sync_copy(in_hbm.at[wid * 2, 1], ibuf)

        out_dma = None
        for bi in range(2):
            b = wid * 2 + bi

            @plsc.parallel_loop(0, P * NB1 // 16, unroll=8)
            def z1(i):
                h1[pl.ds(i * 16, 16)] = zeros16

            @plsc.parallel_loop(0, 32 * NB2 // 16, unroll=8)
            def z2(i):
                h2[pl.ds(i * 16, 16)] = zeros16
                h3[pl.ds(i * 16, 16)] = zeros16

            @plsc.parallel_loop(0, NV, unroll=4)
            def packA(i):
                s = i * 16
                d = dbuf[pl.ds(s, 16)]
                nd = ibuf[pl.ds(s, 16)]
                pid = (nd + 0.5).astype(_i32)
                off = lax.bitcast_convert_type(d, _i32) - C0
                off = jnp.minimum(off, (1 << 24) - (1 << 14))
                valid = (pid >= 1) & (pid <= P) & (d > 3.0)
                k = jnp.where(valid, ((pid - 1) << 24) | off, -1)
                key_v[pl.ds(s, 16)] = k
                pi = (k >> 24) & 7
                bin1 = jnp.minimum((k & 0xFFFFFF) >> 14, NB1 - 1)
                plsc.addupdate_scatter(h1, [pi * NB1 + bin1], ones16,
                                       mask=valid)

            in_dma = None
            if bi == 0:
                nb = wid * 2 + 1
                in_dma = (pltpu.async_copy(in_hbm.at[nb, 0], dbuf, insem),
                          pltpu.async_copy(in_hbm.at[nb, 1], ibuf, insem))

            for p in range(P):
                @plsc.parallel_loop(0, NB1 // 16, unroll=4, carry=zeros16)
                def cs1(j, carry, p=p):
                    sl = pl.ds(p * NB1 + j * 16, 16)
                    s = plsc.cumsum(h1[sl]) + carry
                    h1[sl] = s
                    return _last_lane(s)

            pl8 = jnp.minimum(io, 7)
            n = plsc.load_gather(h1, [pl8 * NB1 + (NB1 - 1)])
            nn = jnp.maximum(n, 1)
            nm1f = (nn - 1).astype(_f32)
            pos25 = 0.25 * nm1f
            pos75 = 0.75 * nm1f
            i0_25 = pos25.astype(_i32)
            i0_75 = pos75.astype(_i32)
            frac25 = pos25 - i0_25.astype(_f32)
            frac75 = pos75 - i0_75.astype(_f32)
            i1_25 = jnp.minimum(i0_25 + 1, nn - 1)
            i1_75 = jnp.minimum(i0_75 + 1, nn - 1)
            lane_p = io & 7
            rA = jnp.where(io < 8, i0_25, _vgather(i1_25, lane_p))
            rB = jnp.where(io < 8, i0_75, _vgather(i1_75, lane_p))

            pbase = lane_p * NB1
            b1A = _bsearch(h1, pbase, rA, 10, NB1)
            b1B = _bsearch(h1, pbase, rB, 10, NB1)
            gpA = plsc.load_gather(h1, [pbase + jnp.maximum(b1A - 1, 0)])
            gpB = plsc.load_gather(h1, [pbase + jnp.maximum(b1B - 1, 0)])
            rA2 = rA - jnp.where(b1A > 0, gpA, 0)
            rB2 = rB - jnp.where(b1B > 0, gpB, 0)
            t1_v[pl.ds(0, 16)] = b1A
            t1_v[pl.ds(16, 16)] = b1B

            @plsc.parallel_loop(0, NV, unroll=4)
            def scanB(i):
                k = key_v[pl.ds(i * 16, 16)]
                m = k >= 0
                pi = (k >> 24) & 7
                off = k & 0xFFFFFF
                b1 = off >> 14
                sub = (off >> 7) & (NB2 - 1)
                for c in range(4):
                    tb = plsc.load_gather(t1_v, [pi + 8 * c])
                    mc = m & (b1 == tb)
                    plsc.addupdate_scatter(
                        h2, [(pi + 8 * c) * NB2 + sub], ones16, mask=mc)

            @plsc.parallel_loop(0, 32, unroll=2)
            def cs2(ch):
                base = ch * NB2
                carry = zeros16
                for kk in range(NB2 // 16):
                    sl = pl.ds(base + kk * 16, 16)
                    s = plsc.cumsum(h2[sl]) + carry
                    h2[sl] = s
                    carry = _last_lane(s)

            cbaseA = io * NB2
            cbaseB = (16 + io) * NB2
            b2A = _bsearch(h2, cbaseA, rA2, 7, NB2)
            b2B = _bsearch(h2, cbaseB, rB2, 7, NB2)
            gp2A = plsc.load_gather(h2, [cbaseA + jnp.maximum(b2A - 1, 0)])
            gp2B = plsc.load_gather(h2, [cbaseB + jnp.maximum(b2B - 1, 0)])
            rA3 = rA2 - jnp.where(b2A > 0, gp2A, 0)
            rB3 = rB2 - jnp.where(b2B > 0, gp2B, 0)
            t12A = (b1A << 7) | b2A
            t12B = (b1B << 7) | b2B
            t12_v[pl.ds(0, 16)] = t12A
            t12_v[pl.ds(16, 16)] = t12B

            @plsc.parallel_loop(0, NV, unroll=4)
            def scanC(i):
                k = key_v[pl.ds(i * 16, 16)]
                m = k >= 0
                pi = (k >> 24) & 7
                off = k & 0xFFFFFF
                hi17 = off >> 7
                sub = off & (NB3 - 1)
                for c in range(4):
                    tb = plsc.load_gather(t12_v, [pi + 8 * c])
                    mc = m & (hi17 == tb)
                    plsc.addupdate_scatter(
                        h3, [(pi + 8 * c) * NB3 + sub], ones16, mask=mc)

            @plsc.parallel_loop(0, 32, unroll=2)
            def cs3(ch):
                base = ch * NB3
                carry = zeros16
                for kk in range(NB3 // 16):
                    sl = pl.ds(base + kk * 16, 16)
                    s = plsc.cumsum(h3[sl]) + carry
                    h3[sl] = s
                    carry = _last_lane(s)

            b3A = _bsearch(h3, cbaseA, rA3, 7, NB3)
            b3B = _bsearch(h3, cbaseB, rB3, 7, NB3)
            offA = (t12A << 7) | b3A
            offB = (t12B << 7) | b3B
            valA = lax.bitcast_convert_type(offA + C0, _f32)
            valB = lax.bitcast_convert_type(offB + C0, _f32)

            hi_idx = jnp.minimum(io + 8, 15)
            v1q1 = _vgather(valA, hi_idx)
            v1q3 = _vgather(valB, hi_idx)
            q1 = valA * (1.0 - frac25) + v1q1 * frac25
            q3 = valB * (1.0 - frac75) + v1q3 * frac75
            iqr = q3 - q1
            lb = q1 - 1.5 * iqr
            ub = q3 + 1.5 * iqr
            npos = n > 0
            posinf = jnp.full((16,), jnp.inf, _f32)
            lb_v[...] = jnp.where(npos, lb, posinf)
            ub_v[...] = jnp.where(npos, ub, -posinf)

            if out_dma is not None:
                out_dma.wait()
                out_dma = None

            @plsc.parallel_loop(0, OUT_PAD // 16, unroll=8)
            def zs(i):
                stage[pl.ds(i * 16, 16)] = zf16

            @plsc.parallel_loop(0, NV, unroll=4, carry=zeros16)
            def emit(i, counters):
                s = i * 16
                k = key_v[pl.ds(s, 16)]
                m = k >= 0
                pi = (k >> 24) & 7
                z = lax.bitcast_convert_type((k & 0xFFFFFF) + C0, _f32)
                lbg = plsc.load_gather(lb_v, [pi])
                ubg = plsc.load_gather(ub_v, [pi])
                filt = m & (z >= lbg) & (z <= ubg)
                sh = (pi & 3) << 3
                onev = jnp.left_shift(ones16, sh)
                low = pi < 4
                enc1 = jnp.where(filt & low, onev, 0)
                enc2 = jnp.where(filt & (~low), onev, 0)
                cs1v = plsc.cumsum(enc1)
                cs2v = plsc.cumsum(enc2)
                cssel = jnp.where(low, cs1v, cs2v)
                within = (cssel >> sh) & 255
                basec = _vgather(counters, pi)
                rank = basec + within - 1
                emitm = filt & (rank < K)
                col = pi * (K + 1) + rank
                fpix = (s + io).astype(_f32)
                rowf = ((fpix + 0.5) * 0.005).astype(_i32).astype(_f32)
                colf = fpix - rowf * 200.0
                xch = (colf - 100.0) * INV_FX
                ych = (rowf - 75.0) * INV_FY
                plsc.store_scatter(stage, [col], xch * z, mask=emitm)
                plsc.store_scatter(stage, [col + OUTCOLS], ych * z, mask=emitm)
                plsc.store_scatter(stage, [col + 2 * OUTCOLS], z, mask=emitm)
                tot = jnp.where(io < 4, _last_lane(cs1v), _last_lane(cs2v))
                inc = (tot >> ((io & 3) << 3)) & 255
                return counters + jnp.where(io < 8, inc, 0)

            counters = emit

            flagv = jnp.where(counters > 0, 1.0, 0.0).astype(_f32)
            plsc.store_scatter(stage, [pl8 * (K + 1) + K], flagv, mask=io < 8)

            out_dma = pltpu.async_copy(stage, out_hbm.at[b], outsem)
            if in_dma is not None:
                in_dma[0].wait()
                in_dma[1].wait()
        out_dma.wait()

    return sc_kernel


_kernel_fn = _make_kernel(interpret=False)


@jax.jit
def kernel(depth_mask_3C):
    x = depth_mask_3C.reshape(B, 3, N)
    outp = _kernel_fn(x)
    return outp[:, :3 * OUTCOLS].reshape(B, 3, OUTCOLS)


# speedup vs baseline: 119.1724x; 1.0652x over previous
import functools

import numpy as np
import jax
import jax.numpy as jnp
from jax import lax
from jax.experimental import pallas as pl
from jax.experimental.pallas import tpu as pltpu
from jax.experimental.pallas import tpu_sc as plsc

B, H, W = 64, 150, 200
P, K = 8, 512
N = H * W
NV = N // 16
OUTCOLS = P * (K + 1)
OUTROW = 3 * OUTCOLS
STAGE_PAD = 12320
C0 = int(np.float32(3.0).view(np.int32))
NB1 = 448
NB2 = 128
NB3 = 256

_hfov = float(np.deg2rad(81.0))
_vfov = float(np.deg2rad(59.0))
_fx = W / (2.0 * np.tan(_hfov / 2.0))
_fy = H / (2.0 * np.tan(_vfov / 2.0))
INV_FX = float(1.0 / _fx)
INV_FY = float(1.0 / _fy)

_i32 = jnp.int32
_f32 = jnp.float32


def _io():
    return lax.iota(_i32, 16)


def _vgather(x, idx):
    return lax.gather(
        x, idx[:, None],
        dimension_numbers=lax.GatherDimensionNumbers(
            offset_dims=(), collapsed_slice_dims=(0,), start_index_map=(0,)),
        slice_sizes=(1,),
        mode=lax.GatherScatterMode.PROMISE_IN_BOUNDS)


def _last_lane(x):
    return _vgather(x, jnp.full((16,), 15, _i32))


def _bsearch(ref, base, r, iters, nbins, packed_hi=None):
    lo = jnp.zeros((16,), _i32)
    hi = jnp.full((16,), nbins, _i32)
    for _ in range(iters):
        mid = (lo + hi) >> 1
        g = plsc.load_gather(ref, [base + jnp.minimum(mid, nbins - 1)])
        if packed_hi is not None:
            g = jnp.where(packed_hi, lax.shift_right_logical(g, 16),
                          g & 0xFFFF)
        pred = g > r
        hi = jnp.where(pred, mid, hi)
        lo = jnp.where(pred, lo, mid + 1)
    return jnp.minimum(lo, nbins - 1)


def _make_kernel(interpret=False):
    mesh = plsc.VectorSubcoreMesh(core_axis_name="c", subcore_axis_name="s",
                                  num_cores=2, num_subcores=16)

    @functools.partial(
        pl.kernel,
        mesh=mesh,
        out_type=jax.ShapeDtypeStruct((B, STAGE_PAD), _f32),
        scratch_types=[
            pltpu.VMEM((N,), _f32),
            pltpu.VMEM((N,), _f32),
            pltpu.VMEM((N,), _i32),
            pltpu.VMEM((P * NB1,), _i32),
            pltpu.VMEM((2 * P * NB2,), _i32),
            pltpu.VMEM((2 * P * NB3,), _i32),
            pltpu.VMEM((STAGE_PAD,), _f32),
            pltpu.VMEM((32,), _i32),
            pltpu.VMEM((32,), _i32),
            pltpu.VMEM((16,), _f32),
            pltpu.VMEM((16,), _f32),
            pltpu.SemaphoreType.DMA,
            pltpu.SemaphoreType.DMA,
        ],
        compiler_params=pltpu.CompilerParams(needs_layout_passes=False),
        interpret=interpret,
    )
    def sc_kernel(in_hbm, out_hbm, dbuf, ibuf, key_v, h1, h2, h3, stage,
                  t1p_v, t12p_v, lb_v, ub_v, insem, outsem):
        wid = lax.axis_index("s") * 2 + lax.axis_index("c")
        io = _io()
        zeros16 = jnp.zeros((16,), _i32)
        ones16 = jnp.ones((16,), _i32)
        zf16 = jnp.zeros((16,), _f32)

        pltpu.sync_copy(in_hbm.at[wid * 2, 0], dbuf)
        pltpu.sync_copy(in_hbm.at[wid * 2, 1], ibuf)

        out_dma = None
        for bi in range(2):
            b = wid * 2 + bi

            @plsc.parallel_loop(0, P * NB1 // 16, unroll=8)
            def z1(i):
                h1[pl.ds(i * 16, 16)] = zeros16

            @plsc.parallel_loop(0, 2 * P * NB2 // 16, unroll=8)
            def z2(i):
                h2[pl.ds(i * 16, 16)] = zeros16
                h3[pl.ds(i * 16, 16)] = zeros16
                h3[pl.ds(2 * P * NB2 + i * 16, 16)] = zeros16

            @plsc.parallel_loop(0, NV, unroll=8)
            def packA(i):
                s = i * 16
                d = dbuf[pl.ds(s, 16)]
                nd = ibuf[pl.ds(s, 16)]
                pid = (nd + 0.5).astype(_i32)
                off = lax.bitcast_convert_type(d, _i32) - C0
                off = jnp.minimum(off, (1 << 24) - (1 << 15))
                valid = (pid >= 1) & (pid <= P) & (d > 3.0)
                k = jnp.where(valid, ((pid - 1) << 24) | off, -1)
                key_v[pl.ds(s, 16)] = k
                pi = (k >> 24) & 7
                bin1 = jnp.minimum((k & 0xFFFFFF) >> 15, NB1 - 1)
                plsc.addupdate_scatter(h1, [pi * NB1 + bin1], ones16,
                                       mask=valid)

            in_dma = None
            if bi == 0:
                nb = wid * 2 + 1
                in_dma = (pltpu.async_copy(in_hbm.at[nb, 0], dbuf, insem),
                          pltpu.async_copy(in_hbm.at[nb, 1], ibuf, insem))

            for p in range(P):
                @plsc.parallel_loop(0, NB1 // 16, unroll=4, carry=zeros16)
                def cs1(j, carry, p=p):
                    sl = pl.ds(p * NB1 + j * 16, 16)
                    s = plsc.cumsum(h1[sl]) + carry
                    h1[sl] = s
                    return _last_lane(s)

            pl8 = jnp.minimum(io, 7)
            n = plsc.load_gather(h1, [pl8 * NB1 + (NB1 - 1)])
            nn = jnp.maximum(n, 1)
            nm1f = (nn - 1).astype(_f32)
            pos25 = 0.25 * nm1f
            pos75 = 0.75 * nm1f
            i0_25 = pos25.astype(_i32)
            i0_75 = pos75.astype(_i32)
            frac25 = pos25 - i0_25.astype(_f32)
            frac75 = pos75 - i0_75.astype(_f32)
            i1_25 = jnp.minimum(i0_25 + 1, nn - 1)
            i1_75 = jnp.minimum(i0_75 + 1, nn - 1)
            lane_p = io & 7
            rA = jnp.where(io < 8, i0_25, _vgather(i1_25, lane_p))
            rB = jnp.where(io < 8, i0_75, _vgather(i1_75, lane_p))

            pbase = lane_p * NB1
            b1A = _bsearch(h1, pbase, rA, 9, NB1)
            b1B = _bsearch(h1, pbase, rB, 9, NB1)
            gpA = plsc.load_gather(h1, [pbase + jnp.maximum(b1A - 1, 0)])
            gpB = plsc.load_gather(h1, [pbase + jnp.maximum(b1B - 1, 0)])
            rA2 = rA - jnp.where(b1A > 0, gpA, 0)
            rB2 = rB - jnp.where(b1B > 0, gpB, 0)
            hi_idx = jnp.minimum(io + 8, 15)
            t1pA = b1A | (_vgather(b1A, hi_idx) << 16)
            t1pB = b1B | (_vgather(b1B, hi_idx) << 16)
            t1p_v[pl.ds(0, 16)] = t1pA
            t1p_v[pl.ds(16, 16)] = t1pB

            @plsc.parallel_loop(0, NV, unroll=8)
            def scanB(i):
                k = key_v[pl.ds(i * 16, 16)]
                m = k >= 0
                pi = (k >> 24) & 7
                off = k & 0xFFFFFF
                b1 = off >> 15
                sub = (off >> 8) & (NB2 - 1)
                for pair in range(2):
                    tp = plsc.load_gather(t1p_v, [pi + 16 * pair])
                    mlo = b1 == (tp & 0xFFFF)
                    mhi = b1 == lax.shift_right_logical(tp, 16)
                    val = (jnp.where(mlo, 1, 0) | jnp.where(mhi, 1 << 16, 0))
                    plsc.addupdate_scatter(
                        h2, [pair * (P * NB2) + pi * NB2 + sub], val,
                        mask=m & (mlo | mhi))

            @plsc.parallel_loop(0, 2 * P, unroll=2)
            def cs2(ch):
                base = ch * NB2
                carry = zeros16
                for kk in range(NB2 // 16):
                    sl = pl.ds(base + kk * 16, 16)
                    s = plsc.cumsum(h2[sl]) + carry
                    h2[sl] = s
                    carry = _last_lane(s)

            is_hi = io >= 8
            cbaseA = lane_p * NB2
            cbaseB = P * NB2 + lane_p * NB2
            b2A = _bsearch(h2, cbaseA, rA2, 7, NB2, packed_hi=is_hi)
            b2B = _bsearch(h2, cbaseB, rB2, 7, NB2, packed_hi=is_hi)
            gp2A = plsc.load_gather(h2, [cbaseA + jnp.maximum(b2A - 1, 0)])
            gp2B = plsc.load_gather(h2, [cbaseB + jnp.maximum(b2B - 1, 0)])
            gp2A = jnp.where(is_hi, lax.shift_right_logical(gp2A, 16),
                             gp2A & 0xFFFF)
            gp2B = jnp.where(is_hi, lax.shift_right_logical(gp2B, 16),
                             gp2B & 0xFFFF)
            rA3 = rA2 - jnp.where(b2A > 0, gp2A, 0)
            rB3 = rB2 - jnp.where(b2B > 0, gp2B, 0)
            t12A = (b1A << 7) | b2A
            t12B = (b1B << 7) | b2B
            t12pA = t12A | (_vgather(t12A, hi_idx) << 16)
            t12pB = t12B | (_vgather(t12B, hi_idx) << 16)
            t12p_v[pl.ds(0, 16)] = t12pA
            t12p_v[pl.ds(16, 16)] = t12pB

            @plsc.parallel_loop(0, NV, unroll=8)
            def scanC(i):
                k = key_v[pl.ds(i * 16, 16)]
                m = k >= 0
                pi = (k >> 24) & 7
                off = k & 0xFFFFFF
                hi16 = off >> 8
                sub = off & (NB3 - 1)
                for pair in range(2):
                    tp = plsc.load_gather(t12p_v, [pi + 16 * pair])
                    mlo = hi16 == (tp & 0xFFFF)
                    mhi = hi16 == lax.shift_right_logical(tp, 16)
                    val = (jnp.where(mlo, 1, 0) | jnp.where(mhi, 1 << 16, 0))
                    plsc.addupdate_scatter(
                        h3, [pair * (P * NB3) + pi * NB3 + sub], val,
                        mask=m & (mlo | mhi))

            @plsc.parallel_loop(0, 2 * P, unroll=2)
            def cs3(ch):
                base = ch * NB3
                carry = zeros16
                for kk in range(NB3 // 16):
                    sl = pl.ds(base + kk * 16, 16)
                    s = plsc.cumsum(h3[sl]) + carry
                    h3[sl] = s
                    carry = _last_lane(s)

            dbaseA = lane_p * NB3
            dbaseB = P * NB3 + lane_p * NB3
            b3A = _bsearch(h3, dbaseA, rA3, 8, NB3, packed_hi=is_hi)
            b3B = _bsearch(h3, dbaseB, rB3, 8, NB3, packed_hi=is_hi)
            offA = (t12A << 8) | b3A
            offB = (t12B << 8) | b3B
            valA = lax.bitcast_convert_type(offA + C0, _f32)
            valB = lax.bitcast_convert_type(offB + C0, _f32)

            v1q1 = _vgather(valA, hi_idx)
            v1q3 = _vgather(valB, hi_idx)
            q1 = valA * (1.0 - frac25) + v1q1 * frac25
            q3 = valB * (1.0 - frac75) + v1q3 * frac75
            iqr = q3 - q1
            lb = q1 - 1.5 * iqr
            ub = q3 + 1.5 * iqr
            npos = n > 0
            posinf = jnp.full((16,), jnp.inf, _f32)
            lb_v[...] = jnp.where(npos, lb, posinf)
            ub_v[...] = jnp.where(npos, ub, -posinf)

            if out_dma is not None:
                out_dma.wait()
                out_dma = None

            @plsc.parallel_loop(0, STAGE_PAD // 16, unroll=8)
            def zs(i):
                stage[pl.ds(i * 16, 16)] = zf16

            @plsc.parallel_loop(0, NV, unroll=4, carry=zeros16)
            def emit(i, counters):
                s = i * 16
                k = key_v[pl.ds(s, 16)]
                m = k >= 0
                pi = (k >> 24) & 7
                z = lax.bitcast_convert_type((k & 0xFFFFFF) + C0, _f32)
                lbg = plsc.load_gather(lb_v, [pi])
                ubg = plsc.load_gather(ub_v, [pi])
                filt = m & (z >= lbg) & (z <= ubg)
                sh = (pi & 3) << 3
                onev = jnp.left_shift(ones16, sh)
                low = pi < 4
                enc1 = jnp.where(filt & low, onev, 0)
                enc2 = jnp.where(filt & (~low), onev, 0)
                cs1v = plsc.cumsum(enc1)
                cs2v = plsc.cumsum(enc2)
                cssel = jnp.where(low, cs1v, cs2v)
                within = (cssel >> sh) & 255
                basec = _vgather(counters, pi)
                rank = basec + within - 1
                emitm = filt & (rank < K)
                col = pi * (K + 1) + rank
                fpix = (s + io).astype(_f32)
                rowf = ((fpix + 0.5) * 0.005).astype(_i32).astype(_f32)
                colf = fpix - rowf * 200.0
                xch = (colf - 100.0) * INV_FX
                ych = (rowf - 75.0) * INV_FY
                plsc.store_scatter(stage, [col], xch * z, mask=emitm)
                plsc.store_scatter(stage, [col + OUTCOLS], ych * z, mask=emitm)
                plsc.store_scatter(stage, [col + 2 * OUTCOLS], z, mask=emitm)
                tot = jnp.where(io < 4, _last_lane(cs1v), _last_lane(cs2v))
                inc = (tot >> ((io & 3) << 3)) & 255
                return counters + jnp.where(io < 8, inc, 0)

            counters = emit

            flagv = jnp.where(counters > 0, 1.0, 0.0).astype(_f32)
            plsc.store_scatter(stage, [pl8 * (K + 1) + K], flagv, mask=io < 8)

            out_dma = pltpu.async_copy(stage, out_hbm.at[b], outsem)
            if in_dma is not None:
                in_dma[0].wait()
                in_dma[1].wait()
        out_dma.wait()

    return sc_kernel


_kernel_fn = _make_kernel(interpret=False)


@jax.jit
def kernel(depth_mask_3C):
    x = depth_mask_3C.reshape(B, 3, N)
    outp = _kernel_fn(x)
    return outp[:, :OUTROW].reshape(B, 3, OUTCOLS)
